# baseline (speedup 1.0000x reference)
import functools

import jax
import jax.numpy as jnp
from jax import lax
from jax.experimental import pallas as pl
from jax.experimental.pallas import tpu as pltpu
from jax.experimental.pallas import tpu_sc as plsc

_BINS = (-1.1503, -0.6745, -0.3186, 0.0, 0.3186, 0.6745, 1.1503)

_N = 32768
_D = 1024

_NW = 32
_L = 16
_SC_ROWS = 8192
_CHUNK = _SC_ROWS // _NW
_SLAB = 128
_NSLAB = _CHUNK // _SLAB

_TC_ROWS = _N - _SC_ROWS
_BM = 2048
_NB = _TC_ROWS // _BM


def _sc_body(x_hbm, out_hbm, a_v, b_v, r_v, sem_a, sem_b):
    c = lax.axis_index("c")
    s = lax.axis_index("s")
    wid = s * 2 + c
    base = _TC_ROWS + wid * _CHUNK

    bufs = (a_v, b_v)
    sems = (sem_a, sem_b)

    def start(k):
        return pltpu.async_copy(
            x_hbm.at[pl.ds(base + k * _SLAB, _SLAB), pl.ds(0, 128)],
            bufs[k % 2],
            sems[k % 2],
        )

    lane = lax.iota(jnp.int32, _L)
    zero = jnp.zeros((_L,), jnp.int32)

    cp = start(0)
    for k in range(_NSLAB):
        cp.wait()
        if k + 1 < _NSLAB:
            cp = start(k + 1)
        slab = bufs[k % 2]
        for g in range(_SLAB // _L):
            y = plsc.load_gather(slab, [g * _L + lane, zero])
            r = jnp.zeros((_L,), jnp.int32)
            for b in _BINS:
                r = r + (y > jnp.float32(b)).astype(jnp.int32)
            r_v[pl.ds(k * _SLAB + g * _L, _L)] = r

    pltpu.sync_copy(r_v, out_hbm.at[pl.ds(wid * _CHUNK, _CHUNK)])


def _sc_part(xr):
    mesh = plsc.VectorSubcoreMesh(core_axis_name="c", subcore_axis_name="s")
    run = functools.partial(
        pl.kernel,
        mesh=mesh,
        out_type=jax.ShapeDtypeStruct((_SC_ROWS,), jnp.int32),
        scratch_types=[
            pltpu.VMEM((_SLAB, 128), jnp.float32),
            pltpu.VMEM((_SLAB, 128), jnp.float32),
            pltpu.VMEM((_CHUNK,), jnp.int32),
            pltpu.SemaphoreType.DMA,
            pltpu.SemaphoreType.DMA,
        ],
        compiler_params=pltpu.CompilerParams(
            use_tc_tiling_on_sc=True,
            needs_layout_passes=False,
        ),
    )(_sc_body)
    return run(xr)


def _tc_body(x_ref, o_ref):
    y = x_ref[0, :, 0]
    r = jnp.zeros((_BM,), jnp.int32)
    for b in _BINS:
        r = r + (y > jnp.float32(b)).astype(jnp.int32)
    o_ref[0, 0, :] = r


def _tc_part(x):
    out = pl.pallas_call(
        _tc_body,
        grid=(_NB,),
        in_specs=[pl.BlockSpec((1, _BM, 128), lambda i: (0, i, 0))],
        out_specs=pl.BlockSpec((1, 1, _BM), lambda i: (i, 0, 0)),
        out_shape=jax.ShapeDtypeStruct((_NB, 1, _BM), jnp.int32),
    )(x)
    return out.reshape(_TC_ROWS)


def kernel(x):
    xr = x.reshape(2 * _N, _D)
    sc_out = _sc_part(xr)
    tc_out = _tc_part(x)
    return jnp.concatenate([tc_out, sc_out])


# speedup vs baseline: 1.3151x; 1.3151x over previous
import functools

import jax
import jax.numpy as jnp
from jax import lax
from jax.experimental import pallas as pl
from jax.experimental.pallas import tpu as pltpu
from jax.experimental.pallas import tpu_sc as plsc

_BINS = (-1.1503, -0.6745, -0.3186, 0.0, 0.3186, 0.6745, 1.1503)

_N = 32768
_D = 1024
_NW = 32
_CHUNK = _N // _NW
_L = 16
_SLAB = 256
_NSLAB = _CHUNK // _SLAB


def _bucketize_body(x_hbm, out_hbm, a_v, b_v, r_v, sem_a, sem_b):
    c = lax.axis_index("c")
    s = lax.axis_index("s")
    wid = s * 2 + c
    base = wid * _CHUNK

    bufs = (a_v, b_v)
    sems = (sem_a, sem_b)

    def start(k):
        return pltpu.async_copy(
            x_hbm.at[pl.ds(base + k * _SLAB, _SLAB), pl.ds(0, 128)],
            bufs[k % 2],
            sems[k % 2],
        )

    lane = lax.iota(jnp.int32, _L)
    zero = jnp.zeros((_L,), jnp.int32)

    cp = start(0)
    for k in range(_NSLAB):
        cp.wait()
        if k + 1 < _NSLAB:
            cp = start(k + 1)
        slab = bufs[k % 2]

        def group(g, carry):
            y = plsc.load_gather(slab, [g * _L + lane, zero])
            r = jnp.zeros((_L,), jnp.int32)
            for b in _BINS:
                r = r + (y > jnp.float32(b)).astype(jnp.int32)
            r_v[pl.ds(k * _SLAB + g * _L, _L)] = r
            return carry

        lax.fori_loop(0, _SLAB // _L, group, 0)

    pltpu.sync_copy(r_v, out_hbm.at[pl.ds(base, _CHUNK)])


def kernel(x):
    xr = x.reshape(2 * _N, _D)
    mesh = plsc.VectorSubcoreMesh(core_axis_name="c", subcore_axis_name="s")
    run = functools.partial(
        pl.kernel,
        mesh=mesh,
        out_type=jax.ShapeDtypeStruct((_N,), jnp.int32),
        scratch_types=[
            pltpu.VMEM((_SLAB, 128), jnp.float32),
            pltpu.VMEM((_SLAB, 128), jnp.float32),
            pltpu.VMEM((_CHUNK,), jnp.int32),
            pltpu.SemaphoreType.DMA,
            pltpu.SemaphoreType.DMA,
        ],
        compiler_params=pltpu.CompilerParams(
            use_tc_tiling_on_sc=True,
            needs_layout_passes=False,
        ),
    )(_bucketize_body)
    return run(xr)


# speedup vs baseline: 1.6923x; 1.2868x over previous
import functools

import jax
import jax.numpy as jnp
from jax import lax
from jax.experimental import pallas as pl
from jax.experimental.pallas import tpu as pltpu
from jax.experimental.pallas import tpu_sc as plsc

_BINS = (-1.1503, -0.6745, -0.3186, 0.0, 0.3186, 0.6745, 1.1503)

_N = 32768
_NW = 32
_CHUNK = _N // _NW
_L = 16
_G = _CHUNK // 128


def _bucketize_body(x_hbm, out_hbm, idx_v, rows_v, r_v, sem):
    c = lax.axis_index("c")
    s = lax.axis_index("s")
    wid = s * 2 + c
    base = wid * _CHUNK

    lane = lax.iota(jnp.int32, _L)
    zero = jnp.zeros((_L,), jnp.int32)

    def idx_group(g, carry):
        r = base + g * _L + lane
        k = r >> 3
        j = r & 7
        idx_v[g // 8, pl.ds((g % 8) * _L, _L)] = (k << 9) + (j << 3)
        return carry

    lax.fori_loop(0, _CHUNK // _L, idx_group, 0)

    cps = [
        pltpu.async_copy(x_hbm.at[idx_v.at[g]], rows_v.at[g], sem)
        for g in range(_G)
    ]
    for cp in cps:
        cp.wait()

    def group(g, carry):
        y = plsc.load_gather(rows_v, [g // 8 + zero, (g % 8) * _L + lane, zero])
        r = jnp.zeros((_L,), jnp.int32)
        for b in _BINS:
            r = r + (y > jnp.float32(b)).astype(jnp.int32)
        r_v[pl.ds(g * _L, _L)] = r
        return carry

    lax.fori_loop(0, _CHUNK // _L, group, 0)

    pltpu.sync_copy(r_v, out_hbm.at[pl.ds(base, _CHUNK)])


def kernel(x):
    xf = (
        x.reshape(8192, 8, 8, 128)
        .transpose(0, 2, 1, 3)
        .reshape(2 * _N * 64, 16)
    )
    mesh = plsc.VectorSubcoreMesh(core_axis_name="c", subcore_axis_name="s")
    run = functools.partial(
        pl.kernel,
        mesh=mesh,
        out_type=jax.ShapeDtypeStruct((_N,), jnp.int32),
        scratch_types=[
            pltpu.VMEM((_G, 128), jnp.int32),
            pltpu.VMEM((_G, 128, 16), jnp.float32),
            pltpu.VMEM((_CHUNK,), jnp.int32),
            pltpu.SemaphoreType.DMA,
        ],
        compiler_params=pltpu.CompilerParams(
            use_tc_tiling_on_sc=False,
            needs_layout_passes=False,
        ),
    )(_bucketize_body)
    return run(xf)
